# baseline (speedup 1.0000x reference)
import functools

import jax
import jax.numpy as jnp
from jax import lax
from jax.experimental import pallas as pl
from jax.experimental.pallas import tpu as pltpu
from jax.experimental.pallas import tpu_sc as plsc

B, N, C, E = 2, 2048, 128, 4
K = N // 2
NC, NS, L = 2, 16, 16
ROWS_PER_TILE = (B * K) // (NC * NS)
RA = 8
REG = 8
REO = 4


def _score_topk_body(x_ref, w1_ref, b1_ref, w2_ref, b2_ref, w3_ref, idx_ref):
    x = x_ref[0]
    h = jnp.dot(x, w1_ref[...], preferred_element_type=jnp.float32)
    h = jnp.maximum(h + b1_ref[...], 0.0)
    h = jnp.dot(h, w2_ref[...], preferred_element_type=jnp.float32)
    h = jnp.maximum(h + b2_ref[...], 0.0)
    s_col = jnp.dot(h, w3_ref[...], preferred_element_type=jnp.float32)
    u = jax.lax.bitcast_convert_type(s_col, jnp.int32)
    k_col = jnp.where(u >= 0, u, u ^ jnp.int32(0x7FFFFFFF))
    k_row = jax.lax.transpose(k_col, (1, 0))
    i_lane = jax.lax.broadcasted_iota(jnp.int32, (1, N), 1)

    rank = jnp.zeros((1, N), jnp.int32)
    jc = 256
    for c in range(N // jc):
        k_j = jax.lax.slice(k_col, (c * jc, 0), ((c + 1) * jc, 1))
        j_sub = jax.lax.broadcasted_iota(jnp.int32, (jc, 1), 0) + c * jc
        beats = (k_j > k_row) | ((k_j == k_row) & (j_sub < i_lane))
        rank = rank + jnp.sum(beats.astype(jnp.int32), axis=0, keepdims=True)

    parts = []
    rc = 256
    for c in range(K // rc):
        r_sub = jax.lax.broadcasted_iota(jnp.int32, (rc, 1), 0) + c * rc
        eq = rank == r_sub
        parts.append(jnp.sum(jnp.where(eq, i_lane, 0), axis=1, keepdims=True))
    idx_ref[...] = jnp.concatenate(parts, axis=0)[None]


def _score_topk(x, W1, b1, W2, b2, W3):
    return pl.pallas_call(
        _score_topk_body,
        grid=(B,),
        in_specs=[
            pl.BlockSpec((1, N, C), lambda b: (b, 0, 0)),
            pl.BlockSpec((C, 64), lambda b: (0, 0)),
            pl.BlockSpec((1, 64), lambda b: (0, 0)),
            pl.BlockSpec((64, 16), lambda b: (0, 0)),
            pl.BlockSpec((1, 16), lambda b: (0, 0)),
            pl.BlockSpec((16, 1), lambda b: (0, 0)),
        ],
        out_specs=pl.BlockSpec((1, K, 1), lambda b: (b, 0, 0)),
        out_shape=jax.ShapeDtypeStruct((B, K, 1), jnp.int32),
    )(x, W1, b1.reshape(1, 64), W2, b2.reshape(1, 16), W3)


def _gather_body(x_hbm, adj_hbm, edge_hbm, idx_hbm, outf_hbm, outa_hbm, oute_hbm,
                 idx_v, myidx_v, cidx_v, feat_v, arow_v, aout_v, erow_v, eout_v,
                 sem):
    cid = lax.axis_index("c")
    sid = lax.axis_index("s")
    wid = cid * NS + sid
    b = wid // NS
    lr0 = (wid % NS) * ROWS_PER_TILE

    iota = lax.iota(jnp.int32, L)
    pltpu.sync_copy(idx_hbm.at[b], idx_v)
    pltpu.sync_copy(idx_hbm.at[b, pl.ds(lr0, ROWS_PER_TILE)], myidx_v)
    def glob(t, _):
        v = myidx_v[pl.ds(t * L, L)]
        myidx_v[pl.ds(t * L, L)] = v + b * N
        return 0
    lax.fori_loop(0, ROWS_PER_TILE // L, glob, 0)
    def build_cidx(g, _):
        base = plsc.load_gather(idx_v, [g * 4 + (iota >> 2)])
        cidx_v[pl.ds(g * L, L)] = base * 4 + (iota & 3)
        return 0
    lax.fori_loop(0, (K * E) // L, build_cidx, 0)

    pltpu.async_copy(x_hbm.at[myidx_v], feat_v, sem).wait()
    pltpu.sync_copy(feat_v, outf_hbm.at[b, pl.ds(lr0, ROWS_PER_TILE)])

    def adj_chunk(ch, _):
        pltpu.async_copy(adj_hbm.at[myidx_v.at[pl.ds(ch * RA, RA)]],
                         arow_v, sem).wait()
        def arow(r, _):
            rv = jnp.full((L,), r, jnp.int32)
            def agrp(g, _):
                col = idx_v[pl.ds(g * L, L)]
                aout_v[pl.ds(r * K + g * L, L)] = plsc.load_gather(arow_v, [rv, col])
                return 0
            lax.fori_loop(0, K // L, agrp, 0)
            return 0
        lax.fori_loop(0, RA, arow, 0)
        pltpu.sync_copy(aout_v, outa_hbm.at[b, pl.ds((lr0 + ch * RA) * K, RA * K)])
        return 0
    lax.fori_loop(0, ROWS_PER_TILE // RA, adj_chunk, 0)

    def edge_chunk(ch, _):
        pltpu.async_copy(edge_hbm.at[myidx_v.at[pl.ds(ch * REG, REG)]],
                         erow_v, sem).wait()
        def ehalf(h, _):
            def erow(r2, _):
                r = h * REO + r2
                rv = jnp.full((L,), r, jnp.int32)
                def egrp(g, _):
                    col = cidx_v[pl.ds(g * L, L)]
                    eout_v[pl.ds(r2 * (K * E) + g * L, L)] = plsc.load_gather(
                        erow_v, [rv, col])
                    return 0
                lax.fori_loop(0, (K * E) // L, egrp, 0)
                return 0
            lax.fori_loop(0, REO, erow, 0)
            pltpu.sync_copy(
                eout_v,
                oute_hbm.at[b, pl.ds((lr0 + ch * REG + h * REO) * K * E,
                                     REO * K * E)])
            return 0
        lax.fori_loop(0, REG // REO, ehalf, 0)
        return 0
    lax.fori_loop(0, ROWS_PER_TILE // REG, edge_chunk, 0)


def _gather_call():
  return functools.partial(
    pl.kernel,
    mesh=plsc.VectorSubcoreMesh(core_axis_name="c", subcore_axis_name="s"),
    out_type=(
        jax.ShapeDtypeStruct((B, K, C), jnp.float32),
        jax.ShapeDtypeStruct((B, K * K), jnp.float32),
        jax.ShapeDtypeStruct((B, K * K * E), jnp.float32),
    ),
    scratch_types=[
        pltpu.VMEM((K,), jnp.int32),
        pltpu.VMEM((ROWS_PER_TILE,), jnp.int32),
        pltpu.VMEM((K * E,), jnp.int32),
        pltpu.VMEM((ROWS_PER_TILE, C), jnp.float32),
        pltpu.VMEM((RA, N), jnp.float32),
        pltpu.VMEM((RA * K,), jnp.float32),
        pltpu.VMEM((REG, N * E), jnp.float32),
        pltpu.VMEM((REO * K * E,), jnp.float32),
        pltpu.SemaphoreType.DMA,
    ],
    compiler_params=pltpu.CompilerParams(needs_layout_passes=False),
  )


def kernel(x, adjacency, edge_features, superpoint_centroids,
           W1, b1, W2, b2, W3, b3):
    del superpoint_centroids, b3
    idx = _score_topk(x, W1, b1, W2, b2, W3).reshape(B, K)
    gather = _gather_call()(_gather_body)
    pooled_f, pooled_a, pooled_e = gather(
        x.reshape(B * N, C),
        adjacency.reshape(B * N, N),
        edge_features.reshape(B * N, N * E),
        idx,
    )
    return (pooled_f,
            pooled_a.reshape(B, K, K),
            pooled_e.reshape(B, K, K, E))
